# Initial kernel scaffold; baseline (speedup 1.0000x reference)
import jax
import jax.numpy as jnp
from jax.experimental import pallas as pl


def kernel(input_seq, seq_lengths, supports, Wg0, bg0, Wc0, bc0, Wg1, bg1, Wc1, bc1, Wfc, bfc):
    raise NotImplementedError("write your pallas kernel here")


# speedup vs baseline: 4.0083x; 4.0083x over previous
import functools

import jax
import jax.numpy as jnp
from jax.experimental import pallas as pl

N = 207
HID = 64
T = 12
D_IN = 2
NCLS = 5


def _gconv(S, inp, st, Wmi, Wms):
    Y0 = inp @ Wmi[0] + st @ Wms[0]
    Y1 = inp @ Wmi[1] + st @ Wms[1]
    Y2 = inp @ Wmi[2] + st @ Wms[2]
    U = S @ Y2
    Z = S @ (Y1 + 2.0 * U)
    return Y0 - Y2 + Z


def _cell(S, inp, st, Wgi, Wgs, bg, Wci, Wcs, bc):
    val = jax.nn.sigmoid(_gconv(S, inp, st, Wgi, Wgs) + bg)
    r = val[:, :HID]
    u = val[:, HID:]
    c = jnp.tanh(_gconv(S, inp, r * st, Wci, Wcs) + bc)
    return u * st + (1.0 - u) * c


def _dcrnn_kernel(inp_ref, seq_ref, s_ref,
                  wg0i_ref, wg0s_ref, bg0_ref, wc0i_ref, wc0s_ref, bc0_ref,
                  wg1i_ref, wg1s_ref, bg1_ref, wc1i_ref, wc1s_ref, bc1_ref,
                  wfc_ref, bfc_ref, out_ref):
    S = s_ref[...]
    Wg0i = [wg0i_ref[m] for m in range(3)]
    Wg0s = [wg0s_ref[m] for m in range(3)]
    Wc0i = [wc0i_ref[m] for m in range(3)]
    Wc0s = [wc0s_ref[m] for m in range(3)]
    Wg1i = [wg1i_ref[m] for m in range(3)]
    Wg1s = [wg1s_ref[m] for m in range(3)]
    Wc1i = [wc1i_ref[m] for m in range(3)]
    Wc1s = [wc1s_ref[m] for m in range(3)]
    bg0 = bg0_ref[...]
    bc0 = bc0_ref[...]
    bg1 = bg1_ref[...]
    bc1 = bc1_ref[...]
    L = seq_ref[0, 0, 0]

    st0 = jnp.zeros((N, HID), jnp.float32)
    st1 = jnp.zeros((N, HID), jnp.float32)
    last = jnp.zeros((N, HID), jnp.float32)
    for t in range(T):
        xt = inp_ref[0, t]
        st0 = _cell(S, xt, st0, Wg0i, Wg0s, bg0, Wc0i, Wc0s, bc0)
        st1 = _cell(S, st0, st1, Wg1i, Wg1s, bg1, Wc1i, Wc1s, bc1)
        last = jnp.where(L == t + 1, st1, last)

    h = jax.nn.relu(last)
    logits = h @ wfc_ref[...] + bfc_ref[...]
    out_ref[0, 0, :] = jnp.max(logits, axis=0)


def _split_w(W, d_in):
    isz = W.shape[0] // 3
    O = W.shape[1]
    Wm = jnp.transpose(W.reshape(isz, 3, O), (1, 0, 2))
    return Wm[:, :d_in, :], Wm[:, d_in:, :]


@jax.jit
def kernel(input_seq, seq_lengths, supports, Wg0, bg0, Wc0, bc0,
           Wg1, bg1, Wc1, bc1, Wfc, bfc):
    B = input_seq.shape[0]
    S = supports[0]
    Wg0i, Wg0s = _split_w(Wg0, D_IN)
    Wc0i, Wc0s = _split_w(Wc0, D_IN)
    Wg1i, Wg1s = _split_w(Wg1, HID)
    Wc1i, Wc1s = _split_w(Wc1, HID)
    seq = seq_lengths.astype(jnp.int32).reshape(B, 1, 1)

    def c(shape):
        return pl.BlockSpec(shape, lambda b: (0,) * len(shape))

    grid_spec = pl.GridSpec(
        grid=(B,),
        in_specs=[
            pl.BlockSpec((1, T, N, D_IN), lambda b: (b, 0, 0, 0)),
            pl.BlockSpec((1, 1, 1), lambda b: (b, 0, 0)),
            c((N, N)),
            c(Wg0i.shape), c(Wg0s.shape), c((1, 2 * HID)),
            c(Wc0i.shape), c(Wc0s.shape), c((1, HID)),
            c(Wg1i.shape), c(Wg1s.shape), c((1, 2 * HID)),
            c(Wc1i.shape), c(Wc1s.shape), c((1, HID)),
            c((HID, NCLS)), c((1, NCLS)),
        ],
        out_specs=pl.BlockSpec((1, 1, NCLS), lambda b: (b, 0, 0)),
    )
    out = pl.pallas_call(
        _dcrnn_kernel,
        grid_spec=grid_spec,
        out_shape=jax.ShapeDtypeStruct((B, 1, NCLS), jnp.float32),
    )(input_seq, seq, S,
      Wg0i, Wg0s, bg0.reshape(1, -1), Wc0i, Wc0s, bc0.reshape(1, -1),
      Wg1i, Wg1s, bg1.reshape(1, -1), Wc1i, Wc1s, bc1.reshape(1, -1),
      Wfc, bfc.reshape(1, -1))
    return out.reshape(B, NCLS)
